# Initial kernel scaffold; baseline (speedup 1.0000x reference)
import jax
import jax.numpy as jnp
from jax.experimental import pallas as pl


def kernel(input, weight):
    raise NotImplementedError("write your pallas kernel here")


# speedup vs baseline: 3.9480x; 3.9480x over previous
import functools

import jax
import jax.numpy as jnp
from jax import lax
from jax.experimental import pallas as pl
from jax.experimental.pallas import tpu as pltpu
from jax.experimental.pallas import tpu_sc as plsc

DIM = 64
IDX_MINOR = 128
SUB = 4


@functools.lru_cache(maxsize=None)
def _build(n_idx_rows: int, dim: int):
    mesh = plsc.VectorSubcoreMesh(core_axis_name="c", subcore_axis_name="s")
    nc, ns = mesh.num_cores, mesh.num_subcores
    nw = nc * ns
    assert n_idx_rows % (nw * SUB) == 0
    chunks_per_w = n_idx_rows // (nw * SUB)
    rows_per_chunk = SUB * IDX_MINOR
    n_rows = n_idx_rows * IDX_MINOR

    @functools.partial(
        pl.kernel,
        out_type=jax.ShapeDtypeStruct((n_rows, dim), jnp.float32),
        mesh=mesh,
        compiler_params=pltpu.CompilerParams(use_tc_tiling_on_sc=False),
        scratch_types=[
            pltpu.VMEM((SUB, IDX_MINOR), jnp.int32),
            pltpu.VMEM((rows_per_chunk, dim), jnp.float32),
            pltpu.SemaphoreType.DMA,
        ],
    )
    def gather_kernel(idx_hbm, table_hbm, out_hbm, idx_v, rows_v, gsem):
        wid = lax.axis_index("s") * nc + lax.axis_index("c")
        row_base = wid * chunks_per_w * SUB

        @pl.loop(0, chunks_per_w)
        def chunk_loop(i):
            r0 = row_base + i * SUB
            pltpu.sync_copy(idx_hbm.at[pl.ds(r0, SUB)], idx_v)
            copies = [
                pltpu.async_copy(
                    table_hbm.at[idx_v.at[j]],
                    rows_v.at[pl.ds(j * IDX_MINOR, IDX_MINOR)],
                    gsem,
                )
                for j in range(SUB)
            ]
            for cpy in copies:
                cpy.wait()
            pltpu.sync_copy(rows_v, out_hbm.at[pl.ds(r0 * IDX_MINOR, rows_per_chunk)])

    return gather_kernel


def kernel(input, weight):
    idx2d = input.reshape(-1, IDX_MINOR).astype(jnp.int32)
    out = _build(idx2d.shape[0], weight.shape[1])(idx2d, weight)
    return out.reshape(*input.shape, weight.shape[1])


# speedup vs baseline: 4.2420x; 1.0745x over previous
import functools

import jax
import jax.numpy as jnp
from jax import lax
from jax.experimental import pallas as pl
from jax.experimental.pallas import tpu as pltpu
from jax.experimental.pallas import tpu_sc as plsc

IDX_MINOR = 128
SUB = 2
NB = 4


@functools.lru_cache(maxsize=None)
def _build(n_idx_rows: int, dim: int):
    mesh = plsc.VectorSubcoreMesh(core_axis_name="c", subcore_axis_name="s")
    nc, ns = mesh.num_cores, mesh.num_subcores
    nw = nc * ns
    assert n_idx_rows % (nw * SUB * NB) == 0
    idx_rows_per_w = n_idx_rows // nw
    chunks_per_w = idx_rows_per_w // SUB
    groups = chunks_per_w // NB
    rows_per_chunk = SUB * IDX_MINOR
    n_rows = n_idx_rows * IDX_MINOR

    @functools.partial(
        pl.kernel,
        out_type=jax.ShapeDtypeStruct((n_rows, dim), jnp.float32),
        mesh=mesh,
        compiler_params=pltpu.CompilerParams(use_tc_tiling_on_sc=False),
        scratch_types=[
            pltpu.VMEM((idx_rows_per_w, IDX_MINOR), jnp.int32),
            pltpu.VMEM((NB, rows_per_chunk, dim), jnp.float32),
            [pltpu.SemaphoreType.DMA] * NB,
            [pltpu.SemaphoreType.DMA] * NB,
        ],
    )
    def gather_kernel(idx_hbm, table_hbm, out_hbm, idx_v, rows_v, gsems, ssems):
        wid = lax.axis_index("s") * nc + lax.axis_index("c")
        row_base = wid * idx_rows_per_w
        pltpu.sync_copy(idx_hbm.at[pl.ds(row_base, idx_rows_per_w)], idx_v)

        def fire_gathers(ci, b):
            for j in range(SUB):
                pltpu.async_copy(
                    table_hbm.at[idx_v.at[ci * SUB + j]],
                    rows_v.at[b].at[pl.ds(j * IDX_MINOR, IDX_MINOR)],
                    gsems[b],
                )

        def drain_gathers(b):
            for j in range(SUB):
                pltpu.make_async_copy(
                    table_hbm.at[idx_v.at[j]],
                    rows_v.at[b].at[pl.ds(j * IDX_MINOR, IDX_MINOR)],
                    gsems[b],
                ).wait()

        def out_slice(ci):
            return out_hbm.at[pl.ds((row_base + ci * SUB) * IDX_MINOR, rows_per_chunk)]

        for b in range(NB):
            fire_gathers(b, b)

        @pl.loop(0, groups - 1)
        def group_loop(t):
            c0 = t * NB
            for b in range(NB):
                drain_gathers(b)
                pltpu.async_copy(rows_v.at[b], out_slice(c0 + b), ssems[b])
            for b in range(NB):
                pltpu.make_async_copy(rows_v.at[b], out_slice(0), ssems[b]).wait()
                fire_gathers(c0 + NB + b, b)

        c0 = (groups - 1) * NB
        for b in range(NB):
            drain_gathers(b)
            pltpu.async_copy(rows_v.at[b], out_slice(c0 + b), ssems[b])
        for b in range(NB):
            pltpu.make_async_copy(rows_v.at[b], out_slice(0), ssems[b]).wait()

    return gather_kernel


def kernel(input, weight):
    idx2d = input.reshape(-1, IDX_MINOR).astype(jnp.int32)
    out = _build(idx2d.shape[0], weight.shape[1])(idx2d, weight)
    return out.reshape(*input.shape, weight.shape[1])
